# baseline (speedup 1.0000x reference)
import jax
import jax.numpy as jnp
from jax import lax
from jax.experimental import pallas as pl
from jax.experimental.pallas import tpu as pltpu
from jax.experimental.pallas import tpu_sc as plsc

NC = 2
NS = 16
NW = NC * NS
LANES = 16


def _make_gather(Dn, Cn, Bn):
    total = Bn * Dn
    n_per = total // NW
    chunk = min(n_per, 8192)
    n_chunks = n_per // chunk
    log2c = Cn.bit_length() - 1

    mesh = plsc.VectorSubcoreMesh(core_axis_name="c", subcore_axis_name="s")

    def body(w_hbm, idx_hbm, out_hbm, idx_v, out_v, sem):
        wid = lax.axis_index("s") * NC + lax.axis_index("c")
        base = wid * n_per

        def chunk_body(k, _):
            off = base + k * chunk
            pltpu.sync_copy(idx_hbm.at[pl.ds(off, chunk)], idx_v)

            def add_body(j, _):
                dbase = lax.rem(j * LANES, Dn)
                dvec = (dbase + lax.iota(jnp.int32, LANES)) << log2c
                sl = pl.ds(j * LANES, LANES)
                idx_v[sl] = idx_v[sl] + dvec
                return _

            lax.fori_loop(0, chunk // LANES, add_body, None)

            pltpu.async_copy(w_hbm.at[idx_v], out_v, sem).wait()
            pltpu.sync_copy(out_v, out_hbm.at[pl.ds(off, chunk)])
            return _

        lax.fori_loop(0, n_chunks, chunk_body, None)

    return pl.kernel(
        body,
        out_type=jax.ShapeDtypeStruct((total,), jnp.float32),
        mesh=mesh,
        scratch_types=[
            pltpu.VMEM((chunk,), jnp.int32),
            pltpu.VMEM((chunk,), jnp.float32),
            pltpu.SemaphoreType.DMA,
        ],
    )


def kernel(weights, indices):
    Dn, Cn = weights.shape
    Bn, _ = indices.shape
    wflat = weights.reshape(-1)
    iflat = indices.reshape(-1)
    out_flat = _make_gather(Dn, Cn, Bn)(wflat, iflat)
    return out_flat.reshape(Bn, Dn)


# speedup vs baseline: 1.8323x; 1.8323x over previous
import jax
import jax.numpy as jnp
from jax import lax
from jax.experimental import pallas as pl
from jax.experimental.pallas import tpu as pltpu
from jax.experimental.pallas import tpu_sc as plsc

NC = 2
NS = 16
NW = NC * NS
LANES = 16
SUB = 8
LANE = 128


def _make_gather(Dn, Cn, Bn):
    total = Bn * Dn
    n_per = total // NW
    period = SUB * Dn
    chunk = period
    n_chunks = n_per // chunk
    log2c = Cn.bit_length() - 1
    log2l = LANE.bit_length() - 1
    assert n_per % chunk == 0 and Cn == (1 << log2c)
    row_stride = Cn * SUB

    mesh = plsc.VectorSubcoreMesh(core_axis_name="c", subcore_axis_name="s")

    def body(w_hbm, idx_hbm, out_hbm, idx_v, out_v, sem):
        wid = lax.axis_index("s") * NC + lax.axis_index("c")
        base = wid * n_per

        iota = lax.iota(jnp.int32, LANES)
        constvec = ((iota >> 3) * row_stride) + ((iota & 7) * LANE)

        def chunk_body(k, _):
            off = base + k * chunk
            pltpu.sync_copy(idx_hbm.at[pl.ds(off, chunk)], idx_v)

            def addr_body(jj, _):
                j = jj * LANES
                dbase = ((j >> 10) << log2l) + (j & (LANE - 1))
                sl = pl.ds(j, LANES)
                iv = idx_v[sl]
                pw = (((iv >> log2l) << 10) + (iv & (LANE - 1))
                      + (dbase >> 3) * row_stride + constvec)
                idx_v[sl] = pw
                return _

            lax.fori_loop(0, chunk // LANES, addr_body, None)

            pltpu.async_copy(w_hbm.at[idx_v], out_v, sem).wait()
            pltpu.sync_copy(out_v, out_hbm.at[pl.ds(off, chunk)])
            return _

        lax.fori_loop(0, n_chunks, chunk_body, None)

    return pl.kernel(
        body,
        out_type=jax.ShapeDtypeStruct((total,), jnp.float32),
        mesh=mesh,
        scratch_types=[
            pltpu.VMEM((chunk,), jnp.int32),
            pltpu.VMEM((chunk,), jnp.float32),
            pltpu.SemaphoreType.DMA,
        ],
    )


def kernel(weights, indices):
    Dn, Cn = weights.shape
    Bn, _ = indices.shape
    wp = weights.reshape(Dn // SUB, SUB, Cn // LANE, LANE)
    wp = wp.transpose(0, 2, 1, 3).reshape(-1)
    ip = indices.reshape(Bn // SUB, SUB, Dn // LANE, LANE)
    ip = ip.transpose(0, 2, 1, 3).reshape(-1)
    out_phys = _make_gather(Dn, Cn, Bn)(wp, ip)
    out = out_phys.reshape(Bn // SUB, Dn // LANE, SUB, LANE)
    out = out.transpose(0, 2, 1, 3).reshape(Bn, Dn)
    return out


# speedup vs baseline: 2.4659x; 1.3458x over previous
import jax
import jax.numpy as jnp
from jax import lax
from jax.experimental import pallas as pl
from jax.experimental.pallas import tpu as pltpu
from jax.experimental.pallas import tpu_sc as plsc

NC = 2
NS = 16
NW = NC * NS
LANES = 16
SUB = 8
LANE = 128


def _make_gather(Dn, Cn, Bn):
    total = Bn * Dn
    n_per = total // NW
    period = SUB * Dn
    chunk = period
    n_chunks = n_per // chunk
    log2c = Cn.bit_length() - 1
    log2l = LANE.bit_length() - 1
    assert n_per % chunk == 0 and Cn == (1 << log2c) and n_chunks % 2 == 0
    row_stride = Cn * SUB

    mesh = plsc.VectorSubcoreMesh(core_axis_name="c", subcore_axis_name="s")

    def body(w_hbm, idx_hbm, out_hbm,
             idx_v0, idx_v1, out_v0, out_v1, gsem, ssem0, ssem1):
        wid = lax.axis_index("s") * NC + lax.axis_index("c")
        base = wid * n_per

        iota = lax.iota(jnp.int32, LANES)
        constvec = ((iota >> 3) * row_stride) + ((iota & 7) * LANE)

        def load_and_addr(k, idx_b):
            off = base + k * chunk
            pltpu.sync_copy(idx_hbm.at[pl.ds(off, chunk)], idx_b)

            def addr_body(jj, _):
                j = jj * LANES
                dbase = ((j >> 10) << log2l) + (j & (LANE - 1))
                sl = pl.ds(j, LANES)
                iv = idx_b[sl]
                pw = (((iv >> log2l) << 10) + (iv & (LANE - 1))
                      + (dbase >> 3) * row_stride + constvec)
                idx_b[sl] = pw
                return _

            lax.fori_loop(0, chunk // LANES, addr_body, None, unroll=4)

        load_and_addr(0, idx_v0)

        def half_step(kk, k, idx_a, idx_b, out_a, ssem_a, prefetch_k):
            @pl.when(kk > 0)
            def _():
                pltpu.make_async_copy(
                    out_a, out_hbm.at[pl.ds(base, chunk)], ssem_a).wait()
            gather = pltpu.make_async_copy(w_hbm.at[idx_a], out_a, gsem)
            gather.start()

            @pl.when(prefetch_k < n_chunks)
            def _():
                load_and_addr(prefetch_k, idx_b)
            gather.wait()
            pltpu.make_async_copy(
                out_a, out_hbm.at[pl.ds(base + k * chunk, chunk)], ssem_a
            ).start()

        def pair_body(kk, _):
            k = kk * 2
            half_step(kk, k, idx_v0, idx_v1, out_v0, ssem0, k + 1)
            half_step(kk, k + 1, idx_v1, idx_v0, out_v1, ssem1, k + 2)
            return _

        lax.fori_loop(0, n_chunks // 2, pair_body, None)

        pltpu.make_async_copy(
            out_v0, out_hbm.at[pl.ds(base, chunk)], ssem0).wait()
        pltpu.make_async_copy(
            out_v1, out_hbm.at[pl.ds(base, chunk)], ssem1).wait()

    return pl.kernel(
        body,
        out_type=jax.ShapeDtypeStruct((total,), jnp.float32),
        mesh=mesh,
        scratch_types=[
            pltpu.VMEM((chunk,), jnp.int32),
            pltpu.VMEM((chunk,), jnp.int32),
            pltpu.VMEM((chunk,), jnp.float32),
            pltpu.VMEM((chunk,), jnp.float32),
            pltpu.SemaphoreType.DMA,
            pltpu.SemaphoreType.DMA,
            pltpu.SemaphoreType.DMA,
        ],
    )


def kernel(weights, indices):
    Dn, Cn = weights.shape
    Bn, _ = indices.shape
    wp = weights.reshape(Dn // SUB, SUB, Cn // LANE, LANE)
    wp = wp.transpose(0, 2, 1, 3).reshape(-1)
    ip = indices.reshape(Bn // SUB, SUB, Dn // LANE, LANE)
    ip = ip.transpose(0, 2, 1, 3).reshape(-1)
    out_phys = _make_gather(Dn, Cn, Bn)(wp, ip)
    out = out_phys.reshape(Bn // SUB, Dn // LANE, SUB, LANE)
    out = out.transpose(0, 2, 1, 3).reshape(Bn, Dn)
    return out
